# baseline (speedup 1.0000x reference)
import functools

import jax
import jax.numpy as jnp
from jax import lax
from jax.experimental import pallas as pl
from jax.experimental.pallas import tpu as pltpu
from jax.experimental.pallas import tpu_sc as plsc

N = 10000
E = 320000
F = 128
NC = 2
NS = 16
NW = NC * NS
EP = E // NW
CHUNK = 80
NCHUNK = EP // CHUNK
GC = 5
GE = GC * CHUNK
NG = NCHUNK // GC
NP = 10240
RPT = NP // NS
ZR = 64
F32 = jnp.float32
_SC_PARAMS = pltpu.CompilerParams(use_tc_tiling_on_sc=False)


def _mesh():
    return plsc.VectorSubcoreMesh(
        core_axis_name="c", subcore_axis_name="s",
        num_cores=NC, num_subcores=NS)


def _zero_f32(ref, nrows, ncols):
    z = jnp.zeros((16,), F32)

    def body(i, _):
        for b in range(ncols // 16):
            ref[i, pl.ds(b * 16, 16)] = z
        return 0

    lax.fori_loop(0, nrows, body, 0)


def _make_agg_kernel(with_deg):
    out_type = [jax.ShapeDtypeStruct((NC, NP, F), F32)]
    scratch = [
        pltpu.VMEM((GE,), jnp.int32),
        pltpu.VMEM((GE,), jnp.int32),
        pltpu.VMEM((GE,), jnp.int32),
        pltpu.VMEM((GE,), jnp.int32),
        pltpu.VMEM((CHUNK, F), F32),
        pltpu.VMEM((CHUNK, F), F32),
        pltpu.VMEM((ZR, F), F32),
        pltpu.VMEM_SHARED((NP, F), F32),
        pltpu.SemaphoreType.DMA,
        pltpu.SemaphoreType.DMA,
        pltpu.SemaphoreType.DMA,
    ]
    if with_deg:
        out_type.append(jax.ShapeDtypeStruct((NC, NP, 16), F32))
        scratch += [
            pltpu.VMEM((CHUNK, 16), F32),
            pltpu.VMEM((ZR, 16), F32),
            pltpu.VMEM_SHARED((NP, 16), F32),
        ]

    def body(table, src, dst, *refs):
        if with_deg:
            (parts_out, deg_out, gs0, gs1, gd0, gd1, rows0, rows1, zbuf,
             acc_sh, gsem, isem, ssem, ones_v, dbuf, deg_sh) = refs
        else:
            (parts_out, gs0, gs1, gd0, gd1, rows0, rows1, zbuf, acc_sh,
             gsem, isem, ssem) = refs
        rows = (rows0, rows1)
        gsrc = (gs0, gs1)
        gdst = (gd0, gd1)
        c = lax.axis_index("c")
        s = lax.axis_index("s")
        wid = s * NC + c
        ebase = wid * EP

        _zero_f32(zbuf, ZR, F)
        for j in range(RPT // ZR):
            pltpu.sync_copy(zbuf, acc_sh.at[pl.ds(s * RPT + j * ZR, ZR)])
        if with_deg:
            _zero_f32(dbuf, ZR, 16)
            for j in range(RPT // ZR):
                pltpu.sync_copy(dbuf, deg_sh.at[pl.ds(s * RPT + j * ZR, ZR)])
            one = jnp.ones((16,), F32)

            def fill_ones(i, _):
                ones_v[i, :] = one
                return 0

            lax.fori_loop(0, CHUNK, fill_ones, 0)
        plsc.subcore_barrier()

        def fire_idx(g, p):
            pltpu.async_copy(src.at[pl.ds(ebase + g * GE, GE)], gsrc[p], isem)
            pltpu.async_copy(dst.at[pl.ds(ebase + g * GE, GE)], gdst[p], isem)

        def drain_idx(p):
            pltpu.make_async_copy(src.at[pl.ds(0, GE)], gsrc[p], isem).wait()
            pltpu.make_async_copy(dst.at[pl.ds(0, GE)], gdst[p], isem).wait()

        def fire_gather(idx, buf):
            pltpu.async_copy(table.at[idx], buf, gsem)

        def drain_gather(buf):
            pltpu.make_async_copy(table.at[pl.ds(0, CHUNK)], buf, gsem).wait()

        def fire_scatter(idx, buf):
            pltpu.async_copy(buf, acc_sh.at[idx], ssem, add=True)
            if with_deg:
                pltpu.async_copy(ones_v, deg_sh.at[idx], ssem, add=True)

        def drain_scatter():
            pltpu.make_async_copy(
                table.at[pl.ds(0, CHUNK)], rows0, ssem).wait()
            if with_deg:
                pltpu.make_async_copy(
                    deg_out.at[0, pl.ds(0, CHUNK)], ones_v, ssem).wait()

        def emit_group(g, gpar, drain_next, fire_next2, fire_last):
            for i in range(GC):
                kpar = (gpar + i) % 2
                if i == 3 and drain_next:
                    drain_idx(1 - gpar)
                drain_gather(rows[kpar])
                if i == 0:
                    @pl.when(g >= 1)
                    def _():
                        drain_scatter()
                else:
                    drain_scatter()
                if i < GC - 1:
                    fire_gather(gsrc[gpar].at[pl.ds((i + 1) * CHUNK, CHUNK)],
                                rows[1 - kpar])
                elif fire_last:
                    fire_gather(gsrc[1 - gpar].at[pl.ds(0, CHUNK)],
                                rows[1 - kpar])
                fire_scatter(gdst[gpar].at[pl.ds(i * CHUNK, CHUNK)],
                             rows[kpar])
            if fire_next2 == "always":
                fire_idx(g + 2, gpar)
            elif fire_next2 == "guard":
                @pl.when(g + 2 <= NG - 1)
                def _():
                    fire_idx(g + 2, gpar)

        fire_idx(0, 0)
        drain_idx(0)
        fire_idx(1, 1)
        fire_gather(gsrc[0].at[pl.ds(0, CHUNK)], rows0)

        def pair_body(t, _):
            emit_group(2 * t, 0, True, "always", True)
            emit_group(2 * t + 1, 1, True, "guard", True)
            return 0

        lax.fori_loop(0, (NG - 1) // 2, pair_body, 0)
        emit_group(NG - 1, 0, False, None, False)
        drain_scatter()
        plsc.subcore_barrier()

        pltpu.sync_copy(acc_sh.at[pl.ds(s * RPT, RPT)],
                        parts_out.at[c, pl.ds(s * RPT, RPT)])
        if with_deg:
            pltpu.sync_copy(deg_sh.at[pl.ds(s * RPT, RPT)],
                            deg_out.at[c, pl.ds(s * RPT, RPT)])

    return pl.kernel(body, tuple(out_type), mesh=_mesh(),
                     scratch_types=scratch, compiler_params=_SC_PARAMS)


@functools.lru_cache(maxsize=None)
def _get_agg_kernel(with_deg):
    return _make_agg_kernel(with_deg)


def _predictor_sc(A, B, u, v, wp2):
    def body(a_hbm, b_hbm, u_hbm, v_hbm, w_hbm, out,
             u_v, v_v, ra0, ra1, rb0, rb1, w_v, s0, s1, gsem, ssem):
        ras = (ra0, ra1)
        rbs = (rb0, rb1)
        s16 = (s0, s1)
        c = lax.axis_index("c")
        s = lax.axis_index("s")
        wid = s * NC + c
        ebase = wid * EP
        pltpu.sync_copy(u_hbm.at[pl.ds(ebase, EP)], u_v)
        pltpu.sync_copy(v_hbm.at[pl.ds(ebase, EP)], v_v)
        pltpu.sync_copy(w_hbm, w_v)
        wbs = [w_v[pl.ds(b * 16, 16)] for b in range(F // 16)]

        def fire_gathers(k, ra, rb):
            pltpu.async_copy(
                a_hbm.at[u_v.at[pl.ds(k * CHUNK, CHUNK)]], ra, gsem)
            pltpu.async_copy(
                b_hbm.at[v_v.at[pl.ds(k * CHUNK, CHUNK)]], rb, gsem)

        def drain_gathers(ra, rb):
            pltpu.make_async_copy(a_hbm.at[pl.ds(0, CHUNK)], ra, gsem).wait()
            pltpu.make_async_copy(b_hbm.at[pl.ds(0, CHUNK)], rb, gsem).wait()

        def compute(ra, rb, sbuf):
            def edge_body(i, _):
                for q in range(4):
                    e = i * 4 + q
                    acc = jnp.zeros((16,), F32)
                    for b in range(F // 16):
                        av = ra[e, pl.ds(b * 16, 16)]
                        bv = rb[e, pl.ds(b * 16, 16)]
                        acc = acc + jnp.maximum(av + bv, 0.0) * wbs[b]
                    sbuf[e, :] = acc
                return 0

            lax.fori_loop(0, CHUNK // 4, edge_body, 0)

        def drain_store(sbuf):
            pltpu.make_async_copy(out.at[pl.ds(0, CHUNK)], sbuf, ssem).wait()

        fire_gathers(0, ra0, rb0)

        def chunk_body(j, _):
            for h in range(2):
                k = 2 * j + h
                drain_gathers(ras[h], rbs[h])
                fire_gathers(k + 1, ras[1 - h], rbs[1 - h])

                @pl.when(k >= 2)
                def _():
                    drain_store(s16[h])

                compute(ras[h], rbs[h], s16[h])
                pltpu.async_copy(s16[h], out.at[pl.ds(ebase + k * CHUNK,
                                                      CHUNK)], ssem)
            return 0

        lax.fori_loop(0, (NCHUNK - 1) // 2, chunk_body, 0)
        k = NCHUNK - 1
        drain_gathers(ras[0], rbs[0])
        drain_store(s16[0])
        compute(ras[0], rbs[0], s16[0])
        drain_store(s16[1])
        pltpu.sync_copy(s16[0], out.at[pl.ds(ebase + k * CHUNK, CHUNK)])

    return pl.kernel(
        body,
        jax.ShapeDtypeStruct((E, 16), F32),
        mesh=_mesh(),
        scratch_types=[
            pltpu.VMEM((EP,), jnp.int32),
            pltpu.VMEM((EP,), jnp.int32),
            pltpu.VMEM((CHUNK, F), F32),
            pltpu.VMEM((CHUNK, F), F32),
            pltpu.VMEM((CHUNK, F), F32),
            pltpu.VMEM((CHUNK, F), F32),
            pltpu.VMEM((F,), F32),
            pltpu.VMEM((CHUNK, 16), F32),
            pltpu.VMEM((CHUNK, 16), F32),
            pltpu.SemaphoreType.DMA,
            pltpu.SemaphoreType.DMA,
        ],
        compiler_params=_SC_PARAMS,
    )(A, B, u, v, wp2)


_RB = 1000
_RBP = 1024


def _mm_body(x_ref, w_ref, o_ref):
    o_ref[...] = jnp.dot(x_ref[...], w_ref[...], preferred_element_type=F32)


def _tc_matmul(x, w):
    return pl.pallas_call(
        _mm_body,
        grid=(N // _RB,),
        in_specs=[
            pl.BlockSpec((_RB, F), lambda i: (i, 0)),
            pl.BlockSpec((F, F), lambda i: (0, 0)),
        ],
        out_specs=pl.BlockSpec((_RB, F), lambda i: (i, 0)),
        out_shape=jax.ShapeDtypeStruct((N, F), F32),
    )(x, w)


def _deg_from16(d_ref):
    deg = jnp.sum(d_ref[...], axis=(0, 2)) * (1.0 / 16.0)
    return jnp.maximum(deg, 1.0)


def _lay1_body(p_ref, d_ref, b1_ref, w2_ref, o_ref):
    acc = p_ref[0] + p_ref[1]
    deg = _deg_from16(d_ref)
    h = jnp.maximum(acc / deg[:, None] + b1_ref[...], 0.0)
    o_ref[...] = jnp.dot(h, w2_ref[...], preferred_element_type=F32)


def _tc_layer1(parts, deg16, b1, w2):
    return pl.pallas_call(
        _lay1_body,
        grid=(NP // _RBP,),
        in_specs=[
            pl.BlockSpec((NC, _RBP, F), lambda i: (0, i, 0)),
            pl.BlockSpec((NC, _RBP, 16), lambda i: (0, i, 0)),
            pl.BlockSpec((1, F), lambda i: (0, 0)),
            pl.BlockSpec((F, F), lambda i: (0, 0)),
        ],
        out_specs=pl.BlockSpec((_RBP, F), lambda i: (i, 0)),
        out_shape=jax.ShapeDtypeStruct((NP, F), F32),
    )(parts, deg16, b1, w2)


def _lay2_body(p_ref, d_ref, b2_ref, wp1_ref, bp1_ref, a_ref, b_ref):
    acc = p_ref[0] + p_ref[1]
    deg = _deg_from16(d_ref)
    h2 = acc / deg[:, None] + b2_ref[...]
    a_ref[...] = (jnp.dot(h2, wp1_ref[0:F, :], preferred_element_type=F32)
                  + bp1_ref[...])
    b_ref[...] = jnp.dot(h2, wp1_ref[F:2 * F, :], preferred_element_type=F32)


def _tc_layer2(parts, deg16, b2, wp1, bp1):
    return pl.pallas_call(
        _lay2_body,
        grid=(NP // _RBP,),
        in_specs=[
            pl.BlockSpec((NC, _RBP, F), lambda i: (0, i, 0)),
            pl.BlockSpec((NC, _RBP, 16), lambda i: (0, i, 0)),
            pl.BlockSpec((1, F), lambda i: (0, 0)),
            pl.BlockSpec((2 * F, F), lambda i: (0, 0)),
            pl.BlockSpec((1, F), lambda i: (0, 0)),
        ],
        out_specs=[
            pl.BlockSpec((_RBP, F), lambda i: (i, 0)),
            pl.BlockSpec((_RBP, F), lambda i: (i, 0)),
        ],
        out_shape=[
            jax.ShapeDtypeStruct((NP, F), F32),
            jax.ShapeDtypeStruct((NP, F), F32),
        ],
    )(parts, deg16, b2, wp1, bp1)


_EB = 8000


def _red_body(s_ref, bp2_ref, o_ref):
    o_ref[...] = jnp.sum(s_ref[...], axis=1, keepdims=True) + bp2_ref[0, 0]


def _tc_reduce(s16, bp2):
    return pl.pallas_call(
        _red_body,
        grid=(E // _EB,),
        in_specs=[
            pl.BlockSpec((_EB, 16), lambda i: (i, 0)),
            pl.BlockSpec((1, 1), lambda i: (0, 0)),
        ],
        out_specs=pl.BlockSpec((_EB, 1), lambda i: (i, 0)),
        out_shape=jax.ShapeDtypeStruct((E, 1), F32),
    )(s16, bp2)


def kernel(x, edge_index, pred_edge_index, W1, b1, W2, b2, Wp1, bp1, Wp2, bp2):
    src = edge_index[0].astype(jnp.int32)
    dst = edge_index[1].astype(jnp.int32)
    u = pred_edge_index[0].astype(jnp.int32)
    v = pred_edge_index[1].astype(jnp.int32)

    xw = _tc_matmul(x, W1)
    parts1, deg16 = _get_agg_kernel(True)(xw, src, dst)
    hw = _tc_layer1(parts1, deg16, b1.reshape(1, F), W2)
    (parts2,) = _get_agg_kernel(False)(hw, src, dst)
    A, B = _tc_layer2(parts2, deg16, b2.reshape(1, F), Wp1, bp1.reshape(1, F))
    s16 = _predictor_sc(A, B, u, v, Wp2.reshape(F))
    return _tc_reduce(s16, bp2.reshape(1, 1))


# speedup vs baseline: 1.0208x; 1.0208x over previous
import functools

import jax
import jax.numpy as jnp
from jax import lax
from jax.experimental import pallas as pl
from jax.experimental.pallas import tpu as pltpu
from jax.experimental.pallas import tpu_sc as plsc

N = 10000
E = 320000
F = 128
NC = 2
NS = 16
NW = NC * NS
EP = E // NW
CHUNK = 80
NCHUNK = EP // CHUNK
GC = 5
GE = GC * CHUNK
NG = NCHUNK // GC
NP = 10240
RPT = NP // NS
ZR = 64
F32 = jnp.float32
_SC_PARAMS = pltpu.CompilerParams(use_tc_tiling_on_sc=False,
                                  needs_layout_passes=False)


def _mesh():
    return plsc.VectorSubcoreMesh(
        core_axis_name="c", subcore_axis_name="s",
        num_cores=NC, num_subcores=NS)


def _zero_f32(ref, nrows, ncols):
    z = jnp.zeros((16,), F32)

    def body(i, _):
        for b in range(ncols // 16):
            ref[i, pl.ds(b * 16, 16)] = z
        return 0

    lax.fori_loop(0, nrows, body, 0)


def _make_agg_kernel(with_deg):
    out_type = [jax.ShapeDtypeStruct((NC, NP, F), F32)]
    scratch = [
        pltpu.VMEM((GE,), jnp.int32),
        pltpu.VMEM((GE,), jnp.int32),
        pltpu.VMEM((GE,), jnp.int32),
        pltpu.VMEM((GE,), jnp.int32),
        pltpu.VMEM((CHUNK, F), F32),
        pltpu.VMEM((CHUNK, F), F32),
        pltpu.VMEM((ZR, F), F32),
        pltpu.VMEM_SHARED((NP, F), F32),
        pltpu.SemaphoreType.DMA,
        pltpu.SemaphoreType.DMA,
        pltpu.SemaphoreType.DMA,
    ]
    if with_deg:
        out_type.append(jax.ShapeDtypeStruct((NC, NP, 16), F32))
        scratch += [
            pltpu.VMEM((CHUNK, 16), F32),
            pltpu.VMEM((ZR, 16), F32),
            pltpu.VMEM_SHARED((NP, 16), F32),
        ]

    def body(table, src, dst, *refs):
        if with_deg:
            (parts_out, deg_out, gs0, gs1, gd0, gd1, rows0, rows1, zbuf,
             acc_sh, gsem, isem, ssem, ones_v, dbuf, deg_sh) = refs
        else:
            (parts_out, gs0, gs1, gd0, gd1, rows0, rows1, zbuf, acc_sh,
             gsem, isem, ssem) = refs
        rows = (rows0, rows1)
        gsrc = (gs0, gs1)
        gdst = (gd0, gd1)
        c = lax.axis_index("c")
        s = lax.axis_index("s")
        wid = s * NC + c
        ebase = wid * EP

        _zero_f32(zbuf, ZR, F)
        for j in range(RPT // ZR):
            pltpu.sync_copy(zbuf, acc_sh.at[pl.ds(s * RPT + j * ZR, ZR)])
        if with_deg:
            _zero_f32(dbuf, ZR, 16)
            for j in range(RPT // ZR):
                pltpu.sync_copy(dbuf, deg_sh.at[pl.ds(s * RPT + j * ZR, ZR)])
            one = jnp.ones((16,), F32)

            def fill_ones(i, _):
                ones_v[i, :] = one
                return 0

            lax.fori_loop(0, CHUNK, fill_ones, 0)
        plsc.subcore_barrier()

        def fire_idx(g, p):
            pltpu.async_copy(src.at[pl.ds(ebase + g * GE, GE)], gsrc[p], isem)
            pltpu.async_copy(dst.at[pl.ds(ebase + g * GE, GE)], gdst[p], isem)

        def drain_idx(p):
            pltpu.make_async_copy(src.at[pl.ds(0, GE)], gsrc[p], isem).wait()
            pltpu.make_async_copy(dst.at[pl.ds(0, GE)], gdst[p], isem).wait()

        def fire_gather(idx, buf):
            pltpu.async_copy(table.at[idx], buf, gsem)

        def drain_gather(buf):
            pltpu.make_async_copy(table.at[pl.ds(0, CHUNK)], buf, gsem).wait()

        def fire_scatter(idx, buf):
            pltpu.async_copy(buf, acc_sh.at[idx], ssem, add=True)
            if with_deg:
                pltpu.async_copy(ones_v, deg_sh.at[idx], ssem, add=True)

        def drain_scatter():
            pltpu.make_async_copy(
                table.at[pl.ds(0, CHUNK)], rows0, ssem).wait()
            if with_deg:
                pltpu.make_async_copy(
                    deg_out.at[0, pl.ds(0, CHUNK)], ones_v, ssem).wait()

        def emit_group(g, gpar, drain_next, fire_next2, fire_last):
            for i in range(GC):
                kpar = (gpar + i) % 2
                if i == 3 and drain_next:
                    drain_idx(1 - gpar)
                drain_gather(rows[kpar])
                if i == 0:
                    @pl.when(g >= 1)
                    def _():
                        drain_scatter()
                else:
                    drain_scatter()
                if i < GC - 1:
                    fire_gather(gsrc[gpar].at[pl.ds((i + 1) * CHUNK, CHUNK)],
                                rows[1 - kpar])
                elif fire_last:
                    fire_gather(gsrc[1 - gpar].at[pl.ds(0, CHUNK)],
                                rows[1 - kpar])
                fire_scatter(gdst[gpar].at[pl.ds(i * CHUNK, CHUNK)],
                             rows[kpar])
            if fire_next2 == "always":
                fire_idx(g + 2, gpar)
            elif fire_next2 == "guard":
                @pl.when(g + 2 <= NG - 1)
                def _():
                    fire_idx(g + 2, gpar)

        fire_idx(0, 0)
        drain_idx(0)
        fire_idx(1, 1)
        fire_gather(gsrc[0].at[pl.ds(0, CHUNK)], rows0)

        def pair_body(t, _):
            emit_group(2 * t, 0, True, "always", True)
            emit_group(2 * t + 1, 1, True, "guard", True)
            return 0

        lax.fori_loop(0, (NG - 1) // 2, pair_body, 0)
        emit_group(NG - 1, 0, False, None, False)
        drain_scatter()
        plsc.subcore_barrier()

        pltpu.sync_copy(acc_sh.at[pl.ds(s * RPT, RPT)],
                        parts_out.at[c, pl.ds(s * RPT, RPT)])
        if with_deg:
            pltpu.sync_copy(deg_sh.at[pl.ds(s * RPT, RPT)],
                            deg_out.at[c, pl.ds(s * RPT, RPT)])

    return pl.kernel(body, tuple(out_type), mesh=_mesh(),
                     scratch_types=scratch, compiler_params=_SC_PARAMS)


@functools.lru_cache(maxsize=None)
def _get_agg_kernel(with_deg):
    return _make_agg_kernel(with_deg)


def _predictor_sc(A, B, u, v, wp2):
    def body(a_hbm, b_hbm, u_hbm, v_hbm, w_hbm, out,
             u_v, v_v, ra0, ra1, rb0, rb1, w_v, s0, s1, gsem, ssem):
        ras = (ra0, ra1)
        rbs = (rb0, rb1)
        s16 = (s0, s1)
        c = lax.axis_index("c")
        s = lax.axis_index("s")
        wid = s * NC + c
        ebase = wid * EP
        pltpu.sync_copy(u_hbm.at[pl.ds(ebase, EP)], u_v)
        pltpu.sync_copy(v_hbm.at[pl.ds(ebase, EP)], v_v)
        pltpu.sync_copy(w_hbm, w_v)
        wbs = [w_v[pl.ds(b * 16, 16)] for b in range(F // 16)]

        def fire_gathers(k, ra, rb):
            pltpu.async_copy(
                a_hbm.at[u_v.at[pl.ds(k * CHUNK, CHUNK)]], ra, gsem)
            pltpu.async_copy(
                b_hbm.at[v_v.at[pl.ds(k * CHUNK, CHUNK)]], rb, gsem)

        def drain_gathers(ra, rb):
            pltpu.make_async_copy(a_hbm.at[pl.ds(0, CHUNK)], ra, gsem).wait()
            pltpu.make_async_copy(b_hbm.at[pl.ds(0, CHUNK)], rb, gsem).wait()

        himask = jnp.full((16,), 0xFFFF0000, dtype=jnp.uint32)
        shift = jnp.full((16,), 16, dtype=jnp.uint32)

        def widen(row_ref, e, q):
            x = plsc.bitcast(row_ref[e, pl.ds(q * 32, 32)], jnp.uint32)
            lo = plsc.bitcast(x << shift, F32)
            hi = plsc.bitcast(x & himask, F32)
            return lo, hi

        def compute(ra, rb, sbuf):
            def edge_body(i, _):
                for u4 in range(4):
                    e = i * 4 + u4
                    acc = jnp.zeros((16,), F32)
                    for q in range(F // 32):
                        alo, ahi = widen(ra, e, q)
                        blo, bhi = widen(rb, e, q)
                        acc = acc + (jnp.maximum(alo + blo, 0.0) * wbs[2 * q]
                                     + jnp.maximum(ahi + bhi, 0.0)
                                     * wbs[2 * q + 1])
                    sbuf[e, :] = acc
                return 0

            lax.fori_loop(0, CHUNK // 4, edge_body, 0)

        def drain_store(sbuf):
            pltpu.make_async_copy(out.at[pl.ds(0, CHUNK)], sbuf, ssem).wait()

        fire_gathers(0, ra0, rb0)

        def chunk_body(j, _):
            for h in range(2):
                k = 2 * j + h
                drain_gathers(ras[h], rbs[h])
                fire_gathers(k + 1, ras[1 - h], rbs[1 - h])

                @pl.when(k >= 2)
                def _():
                    drain_store(s16[h])

                compute(ras[h], rbs[h], s16[h])
                pltpu.async_copy(s16[h], out.at[pl.ds(ebase + k * CHUNK,
                                                      CHUNK)], ssem)
            return 0

        lax.fori_loop(0, (NCHUNK - 1) // 2, chunk_body, 0)
        k = NCHUNK - 1
        drain_gathers(ras[0], rbs[0])
        drain_store(s16[0])
        compute(ras[0], rbs[0], s16[0])
        drain_store(s16[1])
        pltpu.sync_copy(s16[0], out.at[pl.ds(ebase + k * CHUNK, CHUNK)])

    return pl.kernel(
        body,
        jax.ShapeDtypeStruct((E, 16), F32),
        mesh=_mesh(),
        scratch_types=[
            pltpu.VMEM((EP,), jnp.int32),
            pltpu.VMEM((EP,), jnp.int32),
            pltpu.VMEM((CHUNK, F), jnp.bfloat16),
            pltpu.VMEM((CHUNK, F), jnp.bfloat16),
            pltpu.VMEM((CHUNK, F), jnp.bfloat16),
            pltpu.VMEM((CHUNK, F), jnp.bfloat16),
            pltpu.VMEM((F,), F32),
            pltpu.VMEM((CHUNK, 16), F32),
            pltpu.VMEM((CHUNK, 16), F32),
            pltpu.SemaphoreType.DMA,
            pltpu.SemaphoreType.DMA,
        ],
        compiler_params=_SC_PARAMS,
    )(A, B, u, v, wp2)


_RB = 1000
_RBP = 1024


def _mm_body(x_ref, w_ref, o_ref):
    o_ref[...] = jnp.dot(x_ref[...], w_ref[...], preferred_element_type=F32)


def _tc_matmul(x, w):
    return pl.pallas_call(
        _mm_body,
        grid=(N // _RB,),
        in_specs=[
            pl.BlockSpec((_RB, F), lambda i: (i, 0)),
            pl.BlockSpec((F, F), lambda i: (0, 0)),
        ],
        out_specs=pl.BlockSpec((_RB, F), lambda i: (i, 0)),
        out_shape=jax.ShapeDtypeStruct((N, F), F32),
    )(x, w)


def _deg_from16(d_ref):
    deg = jnp.sum(d_ref[...], axis=(0, 2)) * (1.0 / 16.0)
    return jnp.maximum(deg, 1.0)


def _lay1_body(p_ref, d_ref, b1_ref, w2_ref, o_ref):
    acc = p_ref[0] + p_ref[1]
    deg = _deg_from16(d_ref)
    h = jnp.maximum(acc / deg[:, None] + b1_ref[...], 0.0)
    o_ref[...] = jnp.dot(h, w2_ref[...], preferred_element_type=F32)


def _tc_layer1(parts, deg16, b1, w2):
    return pl.pallas_call(
        _lay1_body,
        grid=(NP // _RBP,),
        in_specs=[
            pl.BlockSpec((NC, _RBP, F), lambda i: (0, i, 0)),
            pl.BlockSpec((NC, _RBP, 16), lambda i: (0, i, 0)),
            pl.BlockSpec((1, F), lambda i: (0, 0)),
            pl.BlockSpec((F, F), lambda i: (0, 0)),
        ],
        out_specs=pl.BlockSpec((_RBP, F), lambda i: (i, 0)),
        out_shape=jax.ShapeDtypeStruct((NP, F), F32),
    )(parts, deg16, b1, w2)


def _lay2_body(p_ref, d_ref, b2_ref, wp1_ref, bp1_ref, a_ref, b_ref):
    acc = p_ref[0] + p_ref[1]
    deg = _deg_from16(d_ref)
    h2 = acc / deg[:, None] + b2_ref[...]
    a_ref[...] = (jnp.dot(h2, wp1_ref[0:F, :], preferred_element_type=F32)
                  + bp1_ref[...]).astype(jnp.bfloat16)
    b_ref[...] = jnp.dot(h2, wp1_ref[F:2 * F, :],
                         preferred_element_type=F32).astype(jnp.bfloat16)


def _tc_layer2(parts, deg16, b2, wp1, bp1):
    return pl.pallas_call(
        _lay2_body,
        grid=(NP // _RBP,),
        in_specs=[
            pl.BlockSpec((NC, _RBP, F), lambda i: (0, i, 0)),
            pl.BlockSpec((NC, _RBP, 16), lambda i: (0, i, 0)),
            pl.BlockSpec((1, F), lambda i: (0, 0)),
            pl.BlockSpec((2 * F, F), lambda i: (0, 0)),
            pl.BlockSpec((1, F), lambda i: (0, 0)),
        ],
        out_specs=[
            pl.BlockSpec((_RBP, F), lambda i: (i, 0)),
            pl.BlockSpec((_RBP, F), lambda i: (i, 0)),
        ],
        out_shape=[
            jax.ShapeDtypeStruct((NP, F), jnp.bfloat16),
            jax.ShapeDtypeStruct((NP, F), jnp.bfloat16),
        ],
    )(parts, deg16, b2, wp1, bp1)


_EB = 8000


def _red_body(s_ref, bp2_ref, o_ref):
    o_ref[...] = jnp.sum(s_ref[...], axis=1, keepdims=True) + bp2_ref[0, 0]


def _tc_reduce(s16, bp2):
    return pl.pallas_call(
        _red_body,
        grid=(E // _EB,),
        in_specs=[
            pl.BlockSpec((_EB, 16), lambda i: (i, 0)),
            pl.BlockSpec((1, 1), lambda i: (0, 0)),
        ],
        out_specs=pl.BlockSpec((_EB, 1), lambda i: (i, 0)),
        out_shape=jax.ShapeDtypeStruct((E, 1), F32),
    )(s16, bp2)


def kernel(x, edge_index, pred_edge_index, W1, b1, W2, b2, Wp1, bp1, Wp2, bp2):
    src = edge_index[0].astype(jnp.int32)
    dst = edge_index[1].astype(jnp.int32)
    u = pred_edge_index[0].astype(jnp.int32)
    v = pred_edge_index[1].astype(jnp.int32)

    xw = _tc_matmul(x, W1)
    parts1, deg16 = _get_agg_kernel(True)(xw, src, dst)
    hw = _tc_layer1(parts1, deg16, b1.reshape(1, F), W2)
    (parts2,) = _get_agg_kernel(False)(hw, src, dst)
    A, B = _tc_layer2(parts2, deg16, b2.reshape(1, F), Wp1, bp1.reshape(1, F))
    wp2r = Wp2.reshape(F // 32, 16, 2).transpose(0, 2, 1).reshape(F)
    s16 = _predictor_sc(A, B, u, v, wp2r)
    return _tc_reduce(s16, bp2.reshape(1, 1))


# speedup vs baseline: 1.3964x; 1.3680x over previous
import functools

import jax
import jax.numpy as jnp
from jax import lax
from jax.experimental import pallas as pl
from jax.experimental.pallas import tpu as pltpu
from jax.experimental.pallas import tpu_sc as plsc

N = 10000
E = 320000
F = 128
NC = 2
NS = 16
NW = NC * NS
EP = E // NW
CHUNK = 80
NCHUNK = EP // CHUNK
GC = 5
GE = GC * CHUNK
NG = NCHUNK // GC
NP = 10240
RPT = NP // NS
ZR = 64
F32 = jnp.float32
_SC_PARAMS = pltpu.CompilerParams(use_tc_tiling_on_sc=False,
                                  needs_layout_passes=False)


def _mesh():
    return plsc.VectorSubcoreMesh(
        core_axis_name="c", subcore_axis_name="s",
        num_cores=NC, num_subcores=NS)


def _zero_f32(ref, nrows, ncols):
    z = jnp.zeros((16,), F32)

    def body(i, _):
        for b in range(ncols // 16):
            ref[i, pl.ds(b * 16, 16)] = z
        return 0

    lax.fori_loop(0, nrows, body, 0)


def _make_agg_kernel(with_deg):
    out_type = [jax.ShapeDtypeStruct((NC, NP, F), F32)]
    scratch = [
        pltpu.VMEM((GE,), jnp.int32),
        pltpu.VMEM((GE,), jnp.int32),
        pltpu.VMEM((GE,), jnp.int32),
        pltpu.VMEM((GE,), jnp.int32),
        pltpu.VMEM((CHUNK, F), F32),
        pltpu.VMEM((CHUNK, F), F32),
        pltpu.VMEM((ZR, F), F32),
        pltpu.VMEM_SHARED((NP, F), F32),
        pltpu.SemaphoreType.DMA,
        pltpu.SemaphoreType.DMA,
        pltpu.SemaphoreType.DMA,
    ]
    if with_deg:
        out_type.append(jax.ShapeDtypeStruct((NC, NP, 16), F32))
        scratch += [
            pltpu.VMEM((CHUNK, 16), F32),
            pltpu.VMEM((ZR, 16), F32),
            pltpu.VMEM_SHARED((NP, 16), F32),
        ]

    def body(table, src, dst, *refs):
        if with_deg:
            (parts_out, deg_out, gs0, gs1, gd0, gd1, rows0, rows1, zbuf,
             acc_sh, gsem, isem, ssem, ones_v, dbuf, deg_sh) = refs
        else:
            (parts_out, gs0, gs1, gd0, gd1, rows0, rows1, zbuf, acc_sh,
             gsem, isem, ssem) = refs
        rows = (rows0, rows1)
        gsrc = (gs0, gs1)
        gdst = (gd0, gd1)
        c = lax.axis_index("c")
        s = lax.axis_index("s")
        wid = s * NC + c
        ebase = wid * EP

        _zero_f32(zbuf, ZR, F)
        for j in range(RPT // ZR):
            pltpu.sync_copy(zbuf, acc_sh.at[pl.ds(s * RPT + j * ZR, ZR)])
        if with_deg:
            _zero_f32(dbuf, ZR, 16)
            for j in range(RPT // ZR):
                pltpu.sync_copy(dbuf, deg_sh.at[pl.ds(s * RPT + j * ZR, ZR)])
            one = jnp.ones((16,), F32)

            def fill_ones(i, _):
                ones_v[i, :] = one
                return 0

            lax.fori_loop(0, CHUNK, fill_ones, 0)
        plsc.subcore_barrier()

        def fire_idx(g, p):
            pltpu.async_copy(src.at[pl.ds(ebase + g * GE, GE)], gsrc[p], isem)
            pltpu.async_copy(dst.at[pl.ds(ebase + g * GE, GE)], gdst[p], isem)

        def drain_idx(p):
            pltpu.make_async_copy(src.at[pl.ds(0, GE)], gsrc[p], isem).wait()
            pltpu.make_async_copy(dst.at[pl.ds(0, GE)], gdst[p], isem).wait()

        def fire_gather(idx, buf):
            pltpu.async_copy(table.at[idx], buf, gsem)

        def drain_gather(buf):
            pltpu.make_async_copy(table.at[pl.ds(0, CHUNK)], buf, gsem).wait()

        def fire_scatter(idx, buf):
            pltpu.async_copy(buf, acc_sh.at[idx], ssem, add=True)
            if with_deg:
                pltpu.async_copy(ones_v, deg_sh.at[idx], ssem, add=True)

        def drain_scatter():
            pltpu.make_async_copy(
                table.at[pl.ds(0, CHUNK)], rows0, ssem).wait()
            if with_deg:
                pltpu.make_async_copy(
                    deg_out.at[0, pl.ds(0, CHUNK)], ones_v, ssem).wait()

        def emit_group(g, gpar, drain_next, fire_next2, fire_last):
            for i in range(GC):
                kpar = (gpar + i) % 2
                if i == 3 and drain_next:
                    drain_idx(1 - gpar)
                drain_gather(rows[kpar])
                if i == 0:
                    @pl.when(g >= 1)
                    def _():
                        drain_scatter()
                else:
                    drain_scatter()
                if i < GC - 1:
                    fire_gather(gsrc[gpar].at[pl.ds((i + 1) * CHUNK, CHUNK)],
                                rows[1 - kpar])
                elif fire_last:
                    fire_gather(gsrc[1 - gpar].at[pl.ds(0, CHUNK)],
                                rows[1 - kpar])
                fire_scatter(gdst[gpar].at[pl.ds(i * CHUNK, CHUNK)],
                             rows[kpar])
            if fire_next2 == "always":
                fire_idx(g + 2, gpar)
            elif fire_next2 == "guard":
                @pl.when(g + 2 <= NG - 1)
                def _():
                    fire_idx(g + 2, gpar)

        fire_idx(0, 0)
        drain_idx(0)
        fire_idx(1, 1)
        fire_gather(gsrc[0].at[pl.ds(0, CHUNK)], rows0)

        def pair_body(t, _):
            emit_group(2 * t, 0, True, "always", True)
            emit_group(2 * t + 1, 1, True, "guard", True)
            return 0

        lax.fori_loop(0, (NG - 1) // 2, pair_body, 0)
        emit_group(NG - 1, 0, False, None, False)
        drain_scatter()
        plsc.subcore_barrier()

        pltpu.sync_copy(acc_sh.at[pl.ds(s * RPT, RPT)],
                        parts_out.at[c, pl.ds(s * RPT, RPT)])
        if with_deg:
            pltpu.sync_copy(deg_sh.at[pl.ds(s * RPT, RPT)],
                            deg_out.at[c, pl.ds(s * RPT, RPT)])

    return pl.kernel(body, tuple(out_type), mesh=_mesh(),
                     scratch_types=scratch, compiler_params=_SC_PARAMS)


@functools.lru_cache(maxsize=None)
def _get_agg_kernel(with_deg):
    return _make_agg_kernel(with_deg)


def _predictor_sc(A, B, u, v, wp2):
    def body(a_hbm, b_hbm, u_hbm, v_hbm, w_hbm, out,
             u_v, v_v, ra0, ra1, rb0, rb1, w_v, s0, s1, gsem, ssem):
        ras = (ra0, ra1)
        rbs = (rb0, rb1)
        s16 = (s0, s1)
        c = lax.axis_index("c")
        s = lax.axis_index("s")
        wid = s * NC + c
        ebase = wid * EP
        pltpu.sync_copy(u_hbm.at[pl.ds(ebase, EP)], u_v)
        pltpu.sync_copy(v_hbm.at[pl.ds(ebase, EP)], v_v)
        pltpu.sync_copy(w_hbm, w_v)
        wbs = [w_v[pl.ds(b * 16, 16)] for b in range(F // 16)]

        def fire_gathers(k, ra, rb):
            pltpu.async_copy(
                a_hbm.at[u_v.at[pl.ds(k * CHUNK, CHUNK)]], ra, gsem)
            pltpu.async_copy(
                b_hbm.at[v_v.at[pl.ds(k * CHUNK, CHUNK)]], rb, gsem)

        def drain_gathers(ra, rb):
            pltpu.make_async_copy(a_hbm.at[pl.ds(0, CHUNK)], ra, gsem).wait()
            pltpu.make_async_copy(b_hbm.at[pl.ds(0, CHUNK)], rb, gsem).wait()

        def compute(ra, rb, sbuf):
            def edge_body(i, _):
                for u4 in range(4):
                    e = i * 4 + u4
                    acc = jnp.zeros((16,), F32)
                    for b in range(F // 16):
                        av = ra[e, pl.ds(b * 16, 16)]
                        bv = rb[e, pl.ds(b * 16, 16)]
                        acc = acc + jnp.maximum(av + bv, 0.0) * wbs[b]
                    sbuf[pl.ds(e * 16, 16)] = acc
                return 0

            lax.fori_loop(0, CHUNK // 4, edge_body, 0)

        def drain_store(sbuf):
            pltpu.make_async_copy(
                out.at[pl.ds(0, CHUNK * 16)], sbuf, ssem).wait()

        fire_gathers(0, ra0, rb0)

        def chunk_body(j, _):
            for h in range(2):
                k = 2 * j + h
                drain_gathers(ras[h], rbs[h])
                fire_gathers(k + 1, ras[1 - h], rbs[1 - h])

                @pl.when(k >= 2)
                def _():
                    drain_store(s16[h])

                compute(ras[h], rbs[h], s16[h])
                pltpu.async_copy(
                    s16[h],
                    out.at[pl.ds((ebase + k * CHUNK) * 16, CHUNK * 16)],
                    ssem)
            return 0

        lax.fori_loop(0, (NCHUNK - 1) // 2, chunk_body, 0)
        k = NCHUNK - 1
        drain_gathers(ras[0], rbs[0])
        drain_store(s16[0])
        compute(ras[0], rbs[0], s16[0])
        drain_store(s16[1])
        pltpu.sync_copy(s16[0],
                        out.at[pl.ds((ebase + k * CHUNK) * 16, CHUNK * 16)])

    return pl.kernel(
        body,
        jax.ShapeDtypeStruct((E * 16,), F32),
        mesh=_mesh(),
        scratch_types=[
            pltpu.VMEM((EP,), jnp.int32),
            pltpu.VMEM((EP,), jnp.int32),
            pltpu.VMEM((CHUNK, F), F32),
            pltpu.VMEM((CHUNK, F), F32),
            pltpu.VMEM((CHUNK, F), F32),
            pltpu.VMEM((CHUNK, F), F32),
            pltpu.VMEM((F,), F32),
            pltpu.VMEM((CHUNK * 16,), F32),
            pltpu.VMEM((CHUNK * 16,), F32),
            pltpu.SemaphoreType.DMA,
            pltpu.SemaphoreType.DMA,
        ],
        compiler_params=_SC_PARAMS,
    )(A, B, u, v, wp2)


_RB = 1000
_RBP = 1024


def _mm_body(x_ref, w_ref, o_ref):
    o_ref[...] = jnp.dot(x_ref[...], w_ref[...], preferred_element_type=F32)


def _tc_matmul(x, w):
    return pl.pallas_call(
        _mm_body,
        grid=(N // _RB,),
        in_specs=[
            pl.BlockSpec((_RB, F), lambda i: (i, 0)),
            pl.BlockSpec((F, F), lambda i: (0, 0)),
        ],
        out_specs=pl.BlockSpec((_RB, F), lambda i: (i, 0)),
        out_shape=jax.ShapeDtypeStruct((N, F), F32),
    )(x, w)


def _deg_from16(d_ref):
    deg = jnp.sum(d_ref[...], axis=(0, 2)) * (1.0 / 16.0)
    return jnp.maximum(deg, 1.0)


def _lay1_body(p_ref, d_ref, b1_ref, w2_ref, o_ref):
    acc = p_ref[0] + p_ref[1]
    deg = _deg_from16(d_ref)
    h = jnp.maximum(acc / deg[:, None] + b1_ref[...], 0.0)
    o_ref[...] = jnp.dot(h, w2_ref[...], preferred_element_type=F32)


def _tc_layer1(parts, deg16, b1, w2):
    return pl.pallas_call(
        _lay1_body,
        grid=(NP // _RBP,),
        in_specs=[
            pl.BlockSpec((NC, _RBP, F), lambda i: (0, i, 0)),
            pl.BlockSpec((NC, _RBP, 16), lambda i: (0, i, 0)),
            pl.BlockSpec((1, F), lambda i: (0, 0)),
            pl.BlockSpec((F, F), lambda i: (0, 0)),
        ],
        out_specs=pl.BlockSpec((_RBP, F), lambda i: (i, 0)),
        out_shape=jax.ShapeDtypeStruct((NP, F), F32),
    )(parts, deg16, b1, w2)


def _lay2_body(p_ref, d_ref, b2_ref, wp1_ref, bp1_ref, a_ref, b_ref):
    acc = p_ref[0] + p_ref[1]
    deg = _deg_from16(d_ref)
    h2 = acc / deg[:, None] + b2_ref[...]
    a_ref[...] = (jnp.dot(h2, wp1_ref[0:F, :], preferred_element_type=F32)
                  + bp1_ref[...])
    b_ref[...] = jnp.dot(h2, wp1_ref[F:2 * F, :], preferred_element_type=F32)


def _tc_layer2(parts, deg16, b2, wp1, bp1):
    return pl.pallas_call(
        _lay2_body,
        grid=(NP // _RBP,),
        in_specs=[
            pl.BlockSpec((NC, _RBP, F), lambda i: (0, i, 0)),
            pl.BlockSpec((NC, _RBP, 16), lambda i: (0, i, 0)),
            pl.BlockSpec((1, F), lambda i: (0, 0)),
            pl.BlockSpec((2 * F, F), lambda i: (0, 0)),
            pl.BlockSpec((1, F), lambda i: (0, 0)),
        ],
        out_specs=[
            pl.BlockSpec((_RBP, F), lambda i: (i, 0)),
            pl.BlockSpec((_RBP, F), lambda i: (i, 0)),
        ],
        out_shape=[
            jax.ShapeDtypeStruct((NP, F), F32),
            jax.ShapeDtypeStruct((NP, F), F32),
        ],
    )(parts, deg16, b2, wp1, bp1)


_ER = E * 16 // F
_EB = 4000


def _red_body(s_ref, bp2_ref, o_ref):
    m = (jnp.arange(F)[:, None] // 16 == jnp.arange(8)[None, :]).astype(F32)
    o_ref[...] = (jnp.dot(s_ref[...], m, preferred_element_type=F32)
                  + bp2_ref[0, 0])


def _tc_reduce(s16, bp2):
    return pl.pallas_call(
        _red_body,
        grid=(_ER // _EB,),
        in_specs=[
            pl.BlockSpec((_EB, F), lambda i: (i, 0)),
            pl.BlockSpec((1, 1), lambda i: (0, 0)),
        ],
        out_specs=pl.BlockSpec((_EB, 8), lambda i: (i, 0)),
        out_shape=jax.ShapeDtypeStruct((_ER, 8), F32),
    )(s16, bp2)


def kernel(x, edge_index, pred_edge_index, W1, b1, W2, b2, Wp1, bp1, Wp2, bp2):
    src = edge_index[0].astype(jnp.int32)
    dst = edge_index[1].astype(jnp.int32)
    u = pred_edge_index[0].astype(jnp.int32)
    v = pred_edge_index[1].astype(jnp.int32)

    xw = _tc_matmul(x, W1)
    parts1, deg16 = _get_agg_kernel(True)(xw, src, dst)
    hw = _tc_layer1(parts1, deg16, b1.reshape(1, F), W2)
    (parts2,) = _get_agg_kernel(False)(hw, src, dst)
    A, B = _tc_layer2(parts2, deg16, b2.reshape(1, F), Wp1, bp1.reshape(1, F))
    s16 = _predictor_sc(A, B, u, v, Wp2.reshape(F)).reshape(_ER, F)
    return _tc_reduce(s16, bp2.reshape(1, 1)).reshape(E, 1)


# speedup vs baseline: 1.3979x; 1.0011x over previous
import functools

import jax
import jax.numpy as jnp
from jax import lax
from jax.experimental import pallas as pl
from jax.experimental.pallas import tpu as pltpu
from jax.experimental.pallas import tpu_sc as plsc

N = 10000
E = 320000
F = 128
NC = 2
NS = 16
NW = NC * NS
EP = E // NW
CHUNK = 80
NCHUNK = EP // CHUNK
GC = 5
GE = GC * CHUNK
NG = NCHUNK // GC
NP = 10240
RPT = NP // NS
ZR = 64
F32 = jnp.float32
_SC_PARAMS = pltpu.CompilerParams(use_tc_tiling_on_sc=False,
                                  needs_layout_passes=False)


def _mesh():
    return plsc.VectorSubcoreMesh(
        core_axis_name="c", subcore_axis_name="s",
        num_cores=NC, num_subcores=NS)


def _zero_f32(ref, nrows, ncols):
    z = jnp.zeros((16,), F32)

    def body(i, _):
        for b in range(ncols // 16):
            ref[i, pl.ds(b * 16, 16)] = z
        return 0

    lax.fori_loop(0, nrows, body, 0)


def _make_agg_kernel(with_deg):
    out_type = [jax.ShapeDtypeStruct((NC, NP, F), F32)]
    scratch = [
        pltpu.VMEM((GE,), jnp.int32),
        pltpu.VMEM((GE,), jnp.int32),
        pltpu.VMEM((GE,), jnp.int32),
        pltpu.VMEM((GE,), jnp.int32),
        pltpu.VMEM((CHUNK, F), F32),
        pltpu.VMEM((CHUNK, F), F32),
        pltpu.VMEM((ZR, F), F32),
        pltpu.VMEM_SHARED((NP, F), F32),
        pltpu.SemaphoreType.DMA,
        pltpu.SemaphoreType.DMA,
        pltpu.SemaphoreType.DMA,
    ]
    if with_deg:
        out_type.append(jax.ShapeDtypeStruct((NC, NP, 16), F32))
        scratch += [
            pltpu.VMEM((CHUNK, 16), F32),
            pltpu.VMEM((ZR, 16), F32),
            pltpu.VMEM_SHARED((NP, 16), F32),
        ]

    def body(table, src, dst, *refs):
        if with_deg:
            (parts_out, deg_out, gs0, gs1, gd0, gd1, rows0, rows1, zbuf,
             acc_sh, gsem, isem, ssem, ones_v, dbuf, deg_sh) = refs
        else:
            (parts_out, gs0, gs1, gd0, gd1, rows0, rows1, zbuf, acc_sh,
             gsem, isem, ssem) = refs
        rows = (rows0, rows1)
        gsrc = (gs0, gs1)
        gdst = (gd0, gd1)
        c = lax.axis_index("c")
        s = lax.axis_index("s")
        wid = s * NC + c
        ebase = wid * EP

        _zero_f32(zbuf, ZR, F)
        for j in range(RPT // ZR):
            pltpu.sync_copy(zbuf, acc_sh.at[pl.ds(s * RPT + j * ZR, ZR)])
        if with_deg:
            _zero_f32(dbuf, ZR, 16)
            for j in range(RPT // ZR):
                pltpu.sync_copy(dbuf, deg_sh.at[pl.ds(s * RPT + j * ZR, ZR)])
            one = jnp.ones((16,), F32)

            def fill_ones(i, _):
                ones_v[i, :] = one
                return 0

            lax.fori_loop(0, CHUNK, fill_ones, 0)
        plsc.subcore_barrier()

        def fire_idx(g, p):
            pltpu.async_copy(src.at[pl.ds(ebase + g * GE, GE)], gsrc[p], isem)
            pltpu.async_copy(dst.at[pl.ds(ebase + g * GE, GE)], gdst[p], isem)

        def drain_idx(p):
            pltpu.make_async_copy(src.at[pl.ds(0, GE)], gsrc[p], isem).wait()
            pltpu.make_async_copy(dst.at[pl.ds(0, GE)], gdst[p], isem).wait()

        def fire_gather(idx, buf):
            pltpu.async_copy(table.at[idx], buf, gsem)

        def drain_gather(buf):
            pltpu.make_async_copy(table.at[pl.ds(0, CHUNK)], buf, gsem).wait()

        def fire_scatter(idx, buf):
            pltpu.async_copy(buf, acc_sh.at[idx], ssem, add=True)
            if with_deg:
                pltpu.async_copy(ones_v, deg_sh.at[idx], ssem, add=True)

        def drain_scatter():
            pltpu.make_async_copy(
                table.at[pl.ds(0, CHUNK)], rows0, ssem).wait()
            if with_deg:
                pltpu.make_async_copy(
                    deg_out.at[0, pl.ds(0, CHUNK)], ones_v, ssem).wait()

        def emit_group(g, gpar, drain_next, fire_next2, fire_last):
            for i in range(GC):
                kpar = (gpar + i) % 2
                if i == 3 and drain_next:
                    drain_idx(1 - gpar)
                drain_gather(rows[kpar])
                if i == 0:
                    @pl.when(g >= 1)
                    def _():
                        drain_scatter()
                else:
                    drain_scatter()
                if i < GC - 1:
                    fire_gather(gsrc[gpar].at[pl.ds((i + 1) * CHUNK, CHUNK)],
                                rows[1 - kpar])
                elif fire_last:
                    fire_gather(gsrc[1 - gpar].at[pl.ds(0, CHUNK)],
                                rows[1 - kpar])
                fire_scatter(gdst[gpar].at[pl.ds(i * CHUNK, CHUNK)],
                             rows[kpar])
            if fire_next2 == "always":
                fire_idx(g + 2, gpar)
            elif fire_next2 == "guard":
                @pl.when(g + 2 <= NG - 1)
                def _():
                    fire_idx(g + 2, gpar)

        fire_idx(0, 0)
        drain_idx(0)
        fire_idx(1, 1)
        fire_gather(gsrc[0].at[pl.ds(0, CHUNK)], rows0)

        def pair_body(t, _):
            emit_group(2 * t, 0, True, "always", True)
            emit_group(2 * t + 1, 1, True, "guard", True)
            return 0

        lax.fori_loop(0, (NG - 1) // 2, pair_body, 0)
        emit_group(NG - 1, 0, False, None, False)
        drain_scatter()
        plsc.subcore_barrier()

        pltpu.sync_copy(acc_sh.at[pl.ds(s * RPT, RPT)],
                        parts_out.at[c, pl.ds(s * RPT, RPT)])
        if with_deg:
            pltpu.sync_copy(deg_sh.at[pl.ds(s * RPT, RPT)],
                            deg_out.at[c, pl.ds(s * RPT, RPT)])

    return pl.kernel(body, tuple(out_type), mesh=_mesh(),
                     scratch_types=scratch, compiler_params=_SC_PARAMS)


@functools.lru_cache(maxsize=None)
def _get_agg_kernel(with_deg):
    return _make_agg_kernel(with_deg)


def _predictor_sc(A, B, u, v, wp2):
    def body(a_hbm, b_hbm, u_hbm, v_hbm, w_hbm, out,
             u_v, v_v, ra0, ra1, rb0, rb1, w_v, s0, s1, gsem, ssem):
        ras = (ra0, ra1)
        rbs = (rb0, rb1)
        s16 = (s0, s1)
        c = lax.axis_index("c")
        s = lax.axis_index("s")
        wid = s * NC + c
        ebase = wid * EP
        pltpu.sync_copy(u_hbm.at[pl.ds(ebase, EP)], u_v)
        pltpu.sync_copy(v_hbm.at[pl.ds(ebase, EP)], v_v)
        pltpu.sync_copy(w_hbm, w_v)
        wbs = [w_v[pl.ds(b * 16, 16)] for b in range(F // 16)]

        def fire_gathers(k, ra, rb):
            pltpu.async_copy(
                a_hbm.at[u_v.at[pl.ds(k * CHUNK, CHUNK)]], ra, gsem)
            pltpu.async_copy(
                b_hbm.at[v_v.at[pl.ds(k * CHUNK, CHUNK)]], rb, gsem)

        def drain_gathers(ra, rb):
            pltpu.make_async_copy(a_hbm.at[pl.ds(0, CHUNK)], ra, gsem).wait()
            pltpu.make_async_copy(b_hbm.at[pl.ds(0, CHUNK)], rb, gsem).wait()

        def compute(ra, rb, sbuf):
            def edge_body(i, _):
                for u4 in range(8):
                    e = i * 8 + u4
                    acc = jnp.zeros((16,), F32)
                    for b in range(F // 16):
                        av = ra[e, pl.ds(b * 16, 16)]
                        bv = rb[e, pl.ds(b * 16, 16)]
                        acc = acc + jnp.maximum(av + bv, 0.0) * wbs[b]
                    sbuf[pl.ds(e * 16, 16)] = acc
                return 0

            lax.fori_loop(0, CHUNK // 8, edge_body, 0)

        def drain_store(sbuf):
            pltpu.make_async_copy(
                out.at[pl.ds(0, CHUNK * 16)], sbuf, ssem).wait()

        fire_gathers(0, ra0, rb0)

        def chunk_body(j, _):
            for h in range(2):
                k = 2 * j + h
                drain_gathers(ras[h], rbs[h])
                fire_gathers(k + 1, ras[1 - h], rbs[1 - h])

                @pl.when(k >= 2)
                def _():
                    drain_store(s16[h])

                compute(ras[h], rbs[h], s16[h])
                pltpu.async_copy(
                    s16[h],
                    out.at[pl.ds((ebase + k * CHUNK) * 16, CHUNK * 16)],
                    ssem)
            return 0

        lax.fori_loop(0, (NCHUNK - 1) // 2, chunk_body, 0)
        k = NCHUNK - 1
        drain_gathers(ras[0], rbs[0])
        drain_store(s16[0])
        compute(ras[0], rbs[0], s16[0])
        drain_store(s16[1])
        pltpu.sync_copy(s16[0],
                        out.at[pl.ds((ebase + k * CHUNK) * 16, CHUNK * 16)])

    return pl.kernel(
        body,
        jax.ShapeDtypeStruct((E * 16,), F32),
        mesh=_mesh(),
        scratch_types=[
            pltpu.VMEM((EP,), jnp.int32),
            pltpu.VMEM((EP,), jnp.int32),
            pltpu.VMEM((CHUNK, F), F32),
            pltpu.VMEM((CHUNK, F), F32),
            pltpu.VMEM((CHUNK, F), F32),
            pltpu.VMEM((CHUNK, F), F32),
            pltpu.VMEM((F,), F32),
            pltpu.VMEM((CHUNK * 16,), F32),
            pltpu.VMEM((CHUNK * 16,), F32),
            pltpu.SemaphoreType.DMA,
            pltpu.SemaphoreType.DMA,
        ],
        compiler_params=_SC_PARAMS,
    )(A, B, u, v, wp2)


_RB = 1000
_RBP = 1024


def _mm_body(x_ref, w_ref, o_ref):
    o_ref[...] = jnp.dot(x_ref[...], w_ref[...], preferred_element_type=F32)


def _tc_matmul(x, w):
    return pl.pallas_call(
        _mm_body,
        grid=(N // _RB,),
        in_specs=[
            pl.BlockSpec((_RB, F), lambda i: (i, 0)),
            pl.BlockSpec((F, F), lambda i: (0, 0)),
        ],
        out_specs=pl.BlockSpec((_RB, F), lambda i: (i, 0)),
        out_shape=jax.ShapeDtypeStruct((N, F), F32),
    )(x, w)


def _deg_from16(d_ref):
    deg = jnp.sum(d_ref[...], axis=(0, 2)) * (1.0 / 16.0)
    return jnp.maximum(deg, 1.0)


def _lay1_body(p_ref, d_ref, b1_ref, w2_ref, o_ref):
    acc = p_ref[0] + p_ref[1]
    deg = _deg_from16(d_ref)
    h = jnp.maximum(acc / deg[:, None] + b1_ref[...], 0.0)
    o_ref[...] = jnp.dot(h, w2_ref[...], preferred_element_type=F32)


def _tc_layer1(parts, deg16, b1, w2):
    return pl.pallas_call(
        _lay1_body,
        grid=(NP // _RBP,),
        in_specs=[
            pl.BlockSpec((NC, _RBP, F), lambda i: (0, i, 0)),
            pl.BlockSpec((NC, _RBP, 16), lambda i: (0, i, 0)),
            pl.BlockSpec((1, F), lambda i: (0, 0)),
            pl.BlockSpec((F, F), lambda i: (0, 0)),
        ],
        out_specs=pl.BlockSpec((_RBP, F), lambda i: (i, 0)),
        out_shape=jax.ShapeDtypeStruct((NP, F), F32),
    )(parts, deg16, b1, w2)


def _lay2_body(p_ref, d_ref, b2_ref, wp1_ref, bp1_ref, a_ref, b_ref):
    acc = p_ref[0] + p_ref[1]
    deg = _deg_from16(d_ref)
    h2 = acc / deg[:, None] + b2_ref[...]
    a_ref[...] = (jnp.dot(h2, wp1_ref[0:F, :], preferred_element_type=F32)
                  + bp1_ref[...])
    b_ref[...] = jnp.dot(h2, wp1_ref[F:2 * F, :], preferred_element_type=F32)


def _tc_layer2(parts, deg16, b2, wp1, bp1):
    return pl.pallas_call(
        _lay2_body,
        grid=(NP // _RBP,),
        in_specs=[
            pl.BlockSpec((NC, _RBP, F), lambda i: (0, i, 0)),
            pl.BlockSpec((NC, _RBP, 16), lambda i: (0, i, 0)),
            pl.BlockSpec((1, F), lambda i: (0, 0)),
            pl.BlockSpec((2 * F, F), lambda i: (0, 0)),
            pl.BlockSpec((1, F), lambda i: (0, 0)),
        ],
        out_specs=[
            pl.BlockSpec((_RBP, F), lambda i: (i, 0)),
            pl.BlockSpec((_RBP, F), lambda i: (i, 0)),
        ],
        out_shape=[
            jax.ShapeDtypeStruct((NP, F), F32),
            jax.ShapeDtypeStruct((NP, F), F32),
        ],
    )(parts, deg16, b2, wp1, bp1)


_ER = E * 16 // F
_EB = 4000


def _red_body(s_ref, bp2_ref, o_ref):
    m = (jnp.arange(F)[:, None] // 16 == jnp.arange(8)[None, :]).astype(F32)
    o_ref[...] = (jnp.dot(s_ref[...], m, preferred_element_type=F32)
                  + bp2_ref[0, 0])


def _tc_reduce(s16, bp2):
    return pl.pallas_call(
        _red_body,
        grid=(_ER // _EB,),
        in_specs=[
            pl.BlockSpec((_EB, F), lambda i: (i, 0)),
            pl.BlockSpec((1, 1), lambda i: (0, 0)),
        ],
        out_specs=pl.BlockSpec((_EB, 8), lambda i: (i, 0)),
        out_shape=jax.ShapeDtypeStruct((_ER, 8), F32),
    )(s16, bp2)


def kernel(x, edge_index, pred_edge_index, W1, b1, W2, b2, Wp1, bp1, Wp2, bp2):
    src = edge_index[0].astype(jnp.int32)
    dst = edge_index[1].astype(jnp.int32)
    u = pred_edge_index[0].astype(jnp.int32)
    v = pred_edge_index[1].astype(jnp.int32)

    xw = _tc_matmul(x, W1)
    parts1, deg16 = _get_agg_kernel(True)(xw, src, dst)
    hw = _tc_layer1(parts1, deg16, b1.reshape(1, F), W2)
    (parts2,) = _get_agg_kernel(False)(hw, src, dst)
    A, B = _tc_layer2(parts2, deg16, b2.reshape(1, F), Wp1, bp1.reshape(1, F))
    s16 = _predictor_sc(A, B, u, v, Wp2.reshape(F)).reshape(_ER, F)
    return _tc_reduce(s16, bp2.reshape(1, 1)).reshape(E, 1)


# speedup vs baseline: 1.4011x; 1.0023x over previous
import functools

import jax
import jax.numpy as jnp
from jax import lax
from jax.experimental import pallas as pl
from jax.experimental.pallas import tpu as pltpu
from jax.experimental.pallas import tpu_sc as plsc

N = 10000
E = 320000
F = 128
NC = 2
NS = 16
NW = NC * NS
EP = E // NW
CHUNK = 80
NCHUNK = EP // CHUNK
GC = 5
GE = GC * CHUNK
NG = NCHUNK // GC
NP = 10240
RPT = NP // NS
ZR = 64
F32 = jnp.float32
_SC_PARAMS = pltpu.CompilerParams(use_tc_tiling_on_sc=False,
                                  needs_layout_passes=False)


def _mesh():
    return plsc.VectorSubcoreMesh(
        core_axis_name="c", subcore_axis_name="s",
        num_cores=NC, num_subcores=NS)


def _zero_f32(ref, nrows, ncols):
    z = jnp.zeros((16,), F32)

    def body(i, _):
        for b in range(ncols // 16):
            ref[i, pl.ds(b * 16, 16)] = z
        return 0

    lax.fori_loop(0, nrows, body, 0)


def _make_agg_kernel(with_deg):
    out_type = [jax.ShapeDtypeStruct((NC, NP, F), F32)]
    scratch = [
        pltpu.VMEM((GE,), jnp.int32),
        pltpu.VMEM((GE,), jnp.int32),
        pltpu.VMEM((GE,), jnp.int32),
        pltpu.VMEM((GE,), jnp.int32),
        pltpu.VMEM((CHUNK, F), F32),
        pltpu.VMEM((CHUNK, F), F32),
        pltpu.VMEM((ZR, F), F32),
        pltpu.VMEM_SHARED((NP, F), F32),
        pltpu.SemaphoreType.DMA,
        pltpu.SemaphoreType.DMA,
        pltpu.SemaphoreType.DMA,
    ]
    if with_deg:
        out_type.append(jax.ShapeDtypeStruct((NC, NP, 16), F32))
        scratch += [
            pltpu.VMEM((CHUNK, 16), F32),
            pltpu.VMEM((ZR, 16), F32),
            pltpu.VMEM_SHARED((NP, 16), F32),
        ]

    def body(table, src, dst, *refs):
        if with_deg:
            (parts_out, deg_out, gs0, gs1, gd0, gd1, rows0, rows1, zbuf,
             acc_sh, gsem, isem, ssem, ones_v, dbuf, deg_sh) = refs
        else:
            (parts_out, gs0, gs1, gd0, gd1, rows0, rows1, zbuf, acc_sh,
             gsem, isem, ssem) = refs
        rows = (rows0, rows1)
        gsrc = (gs0, gs1)
        gdst = (gd0, gd1)
        c = lax.axis_index("c")
        s = lax.axis_index("s")
        wid = s * NC + c
        ebase = wid * EP

        _zero_f32(zbuf, ZR, F)
        for j in range(RPT // ZR):
            pltpu.sync_copy(zbuf, acc_sh.at[pl.ds(s * RPT + j * ZR, ZR)])
        if with_deg:
            _zero_f32(dbuf, ZR, 16)
            for j in range(RPT // ZR):
                pltpu.sync_copy(dbuf, deg_sh.at[pl.ds(s * RPT + j * ZR, ZR)])
            one = jnp.ones((16,), F32)

            def fill_ones(i, _):
                ones_v[i, :] = one
                return 0

            lax.fori_loop(0, CHUNK, fill_ones, 0)
        plsc.subcore_barrier()

        def fire_idx(g, p):
            pltpu.async_copy(src.at[pl.ds(ebase + g * GE, GE)], gsrc[p], isem)
            pltpu.async_copy(dst.at[pl.ds(ebase + g * GE, GE)], gdst[p], isem)

        def drain_idx(p):
            pltpu.make_async_copy(src.at[pl.ds(0, GE)], gsrc[p], isem).wait()
            pltpu.make_async_copy(dst.at[pl.ds(0, GE)], gdst[p], isem).wait()

        def fire_gather(idx, buf):
            pltpu.async_copy(table.at[idx], buf, gsem)

        def drain_gather(buf):
            pltpu.make_async_copy(table.at[pl.ds(0, CHUNK)], buf, gsem).wait()

        def fire_scatter(idx, buf):
            pltpu.async_copy(buf, acc_sh.at[idx], ssem, add=True)
            if with_deg:
                pltpu.async_copy(ones_v, deg_sh.at[idx], ssem, add=True)

        def drain_scatter():
            pltpu.make_async_copy(
                table.at[pl.ds(0, CHUNK)], rows0, ssem).wait()
            if with_deg:
                pltpu.make_async_copy(
                    deg_out.at[0, pl.ds(0, CHUNK)], ones_v, ssem).wait()

        def emit_group(g, gpar, drain_next, fire_next2, fire_last):
            for i in range(GC):
                kpar = (gpar + i) % 2
                if i == 3 and drain_next:
                    drain_idx(1 - gpar)
                drain_gather(rows[kpar])
                if i == 0:
                    @pl.when(g >= 1)
                    def _():
                        drain_scatter()
                else:
                    drain_scatter()
                if i < GC - 1:
                    fire_gather(gsrc[gpar].at[pl.ds((i + 1) * CHUNK, CHUNK)],
                                rows[1 - kpar])
                elif fire_last:
                    fire_gather(gsrc[1 - gpar].at[pl.ds(0, CHUNK)],
                                rows[1 - kpar])
                fire_scatter(gdst[gpar].at[pl.ds(i * CHUNK, CHUNK)],
                             rows[kpar])
            if fire_next2 == "always":
                fire_idx(g + 2, gpar)
            elif fire_next2 == "guard":
                @pl.when(g + 2 <= NG - 1)
                def _():
                    fire_idx(g + 2, gpar)

        fire_idx(0, 0)
        drain_idx(0)
        fire_idx(1, 1)
        fire_gather(gsrc[0].at[pl.ds(0, CHUNK)], rows0)

        def pair_body(t, _):
            emit_group(2 * t, 0, True, "always", True)
            emit_group(2 * t + 1, 1, True, "guard", True)
            return 0

        lax.fori_loop(0, (NG - 1) // 2, pair_body, 0)
        emit_group(NG - 1, 0, False, None, False)
        drain_scatter()
        plsc.subcore_barrier()

        pltpu.sync_copy(acc_sh.at[pl.ds(s * RPT, RPT)],
                        parts_out.at[c, pl.ds(s * RPT, RPT)])
        if with_deg:
            pltpu.sync_copy(deg_sh.at[pl.ds(s * RPT, RPT)],
                            deg_out.at[c, pl.ds(s * RPT, RPT)])

    return pl.kernel(body, tuple(out_type), mesh=_mesh(),
                     scratch_types=scratch, compiler_params=_SC_PARAMS)


@functools.lru_cache(maxsize=None)
def _get_agg_kernel(with_deg):
    return _make_agg_kernel(with_deg)


def _predictor_sc(A, B, u, v, wp2):
    def body(a_hbm, b_hbm, u_hbm, v_hbm, w_hbm, out,
             u_v, v_v, ra0, ra1, rb0, rb1, w_v, s0, s1, gsem, ssem):
        ras = (ra0, ra1)
        rbs = (rb0, rb1)
        s16 = (s0, s1)
        c = lax.axis_index("c")
        s = lax.axis_index("s")
        wid = s * NC + c
        ebase = wid * EP
        pltpu.sync_copy(u_hbm.at[pl.ds(ebase, EP)], u_v)
        pltpu.sync_copy(v_hbm.at[pl.ds(ebase, EP)], v_v)
        pltpu.sync_copy(w_hbm, w_v)
        wbs = [w_v[pl.ds(b * 16, 16)] for b in range(F // 16)]

        def fire_gathers(k, ra, rb):
            pltpu.async_copy(
                a_hbm.at[u_v.at[pl.ds(k * CHUNK, CHUNK)]], ra, gsem)
            pltpu.async_copy(
                b_hbm.at[v_v.at[pl.ds(k * CHUNK, CHUNK)]], rb, gsem)

        def drain_gathers(ra, rb):
            pltpu.make_async_copy(a_hbm.at[pl.ds(0, CHUNK)], ra, gsem).wait()
            pltpu.make_async_copy(b_hbm.at[pl.ds(0, CHUNK)], rb, gsem).wait()

        def compute(ra, rb, sbuf):
            @plsc.parallel_loop(0, CHUNK, step=1, unroll=4)
            def edge_body(e):
                acc = jnp.zeros((16,), F32)
                for b in range(F // 16):
                    av = ra[e, pl.ds(b * 16, 16)]
                    bv = rb[e, pl.ds(b * 16, 16)]
                    acc = acc + jnp.maximum(av + bv, 0.0) * wbs[b]
                sbuf[pl.ds(e * 16, 16)] = acc

        def drain_store(sbuf):
            pltpu.make_async_copy(
                out.at[pl.ds(0, CHUNK * 16)], sbuf, ssem).wait()

        fire_gathers(0, ra0, rb0)

        def chunk_body(j, _):
            for h in range(2):
                k = 2 * j + h
                drain_gathers(ras[h], rbs[h])
                fire_gathers(k + 1, ras[1 - h], rbs[1 - h])

                @pl.when(k >= 2)
                def _():
                    drain_store(s16[h])

                compute(ras[h], rbs[h], s16[h])
                pltpu.async_copy(
                    s16[h],
                    out.at[pl.ds((ebase + k * CHUNK) * 16, CHUNK * 16)],
                    ssem)
            return 0

        lax.fori_loop(0, (NCHUNK - 1) // 2, chunk_body, 0)
        k = NCHUNK - 1
        drain_gathers(ras[0], rbs[0])
        drain_store(s16[0])
        compute(ras[0], rbs[0], s16[0])
        drain_store(s16[1])
        pltpu.sync_copy(s16[0],
                        out.at[pl.ds((ebase + k * CHUNK) * 16, CHUNK * 16)])

    return pl.kernel(
        body,
        jax.ShapeDtypeStruct((E * 16,), F32),
        mesh=_mesh(),
        scratch_types=[
            pltpu.VMEM((EP,), jnp.int32),
            pltpu.VMEM((EP,), jnp.int32),
            pltpu.VMEM((CHUNK, F), F32),
            pltpu.VMEM((CHUNK, F), F32),
            pltpu.VMEM((CHUNK, F), F32),
            pltpu.VMEM((CHUNK, F), F32),
            pltpu.VMEM((F,), F32),
            pltpu.VMEM((CHUNK * 16,), F32),
            pltpu.VMEM((CHUNK * 16,), F32),
            pltpu.SemaphoreType.DMA,
            pltpu.SemaphoreType.DMA,
        ],
        compiler_params=_SC_PARAMS,
    )(A, B, u, v, wp2)


_RB = 1000
_RBP = 1024


def _mm_body(x_ref, w_ref, o_ref):
    o_ref[...] = jnp.dot(x_ref[...], w_ref[...], preferred_element_type=F32)


def _tc_matmul(x, w):
    return pl.pallas_call(
        _mm_body,
        grid=(N // _RB,),
        in_specs=[
            pl.BlockSpec((_RB, F), lambda i: (i, 0)),
            pl.BlockSpec((F, F), lambda i: (0, 0)),
        ],
        out_specs=pl.BlockSpec((_RB, F), lambda i: (i, 0)),
        out_shape=jax.ShapeDtypeStruct((N, F), F32),
    )(x, w)


def _deg_from16(d_ref):
    deg = jnp.sum(d_ref[...], axis=(0, 2)) * (1.0 / 16.0)
    return jnp.maximum(deg, 1.0)


def _lay1_body(p_ref, d_ref, b1_ref, w2_ref, o_ref):
    acc = p_ref[0] + p_ref[1]
    deg = _deg_from16(d_ref)
    h = jnp.maximum(acc / deg[:, None] + b1_ref[...], 0.0)
    o_ref[...] = jnp.dot(h, w2_ref[...], preferred_element_type=F32)


def _tc_layer1(parts, deg16, b1, w2):
    return pl.pallas_call(
        _lay1_body,
        grid=(NP // _RBP,),
        in_specs=[
            pl.BlockSpec((NC, _RBP, F), lambda i: (0, i, 0)),
            pl.BlockSpec((NC, _RBP, 16), lambda i: (0, i, 0)),
            pl.BlockSpec((1, F), lambda i: (0, 0)),
            pl.BlockSpec((F, F), lambda i: (0, 0)),
        ],
        out_specs=pl.BlockSpec((_RBP, F), lambda i: (i, 0)),
        out_shape=jax.ShapeDtypeStruct((NP, F), F32),
    )(parts, deg16, b1, w2)


def _lay2_body(p_ref, d_ref, b2_ref, wp1_ref, bp1_ref, a_ref, b_ref):
    acc = p_ref[0] + p_ref[1]
    deg = _deg_from16(d_ref)
    h2 = acc / deg[:, None] + b2_ref[...]
    a_ref[...] = (jnp.dot(h2, wp1_ref[0:F, :], preferred_element_type=F32)
                  + bp1_ref[...])
    b_ref[...] = jnp.dot(h2, wp1_ref[F:2 * F, :], preferred_element_type=F32)


def _tc_layer2(parts, deg16, b2, wp1, bp1):
    return pl.pallas_call(
        _lay2_body,
        grid=(NP // _RBP,),
        in_specs=[
            pl.BlockSpec((NC, _RBP, F), lambda i: (0, i, 0)),
            pl.BlockSpec((NC, _RBP, 16), lambda i: (0, i, 0)),
            pl.BlockSpec((1, F), lambda i: (0, 0)),
            pl.BlockSpec((2 * F, F), lambda i: (0, 0)),
            pl.BlockSpec((1, F), lambda i: (0, 0)),
        ],
        out_specs=[
            pl.BlockSpec((_RBP, F), lambda i: (i, 0)),
            pl.BlockSpec((_RBP, F), lambda i: (i, 0)),
        ],
        out_shape=[
            jax.ShapeDtypeStruct((NP, F), F32),
            jax.ShapeDtypeStruct((NP, F), F32),
        ],
    )(parts, deg16, b2, wp1, bp1)


_ER = E * 16 // F
_EB = 4000


def _red_body(s_ref, bp2_ref, o_ref):
    m = (jnp.arange(F)[:, None] // 16 == jnp.arange(8)[None, :]).astype(F32)
    o_ref[...] = (jnp.dot(s_ref[...], m, preferred_element_type=F32)
                  + bp2_ref[0, 0])


def _tc_reduce(s16, bp2):
    return pl.pallas_call(
        _red_body,
        grid=(_ER // _EB,),
        in_specs=[
            pl.BlockSpec((_EB, F), lambda i: (i, 0)),
            pl.BlockSpec((1, 1), lambda i: (0, 0)),
        ],
        out_specs=pl.BlockSpec((_EB, 8), lambda i: (i, 0)),
        out_shape=jax.ShapeDtypeStruct((_ER, 8), F32),
    )(s16, bp2)


def kernel(x, edge_index, pred_edge_index, W1, b1, W2, b2, Wp1, bp1, Wp2, bp2):
    src = edge_index[0].astype(jnp.int32)
    dst = edge_index[1].astype(jnp.int32)
    u = pred_edge_index[0].astype(jnp.int32)
    v = pred_edge_index[1].astype(jnp.int32)

    xw = _tc_matmul(x, W1)
    parts1, deg16 = _get_agg_kernel(True)(xw, src, dst)
    hw = _tc_layer1(parts1, deg16, b1.reshape(1, F), W2)
    (parts2,) = _get_agg_kernel(False)(hw, src, dst)
    A, B = _tc_layer2(parts2, deg16, b2.reshape(1, F), Wp1, bp1.reshape(1, F))
    s16 = _predictor_sc(A, B, u, v, Wp2.reshape(F)).reshape(_ER, F)
    return _tc_reduce(s16, bp2.reshape(1, 1)).reshape(E, 1)


# speedup vs baseline: 1.4235x; 1.0160x over previous
import functools

import jax
import jax.numpy as jnp
from jax import lax
from jax.experimental import pallas as pl
from jax.experimental.pallas import tpu as pltpu
from jax.experimental.pallas import tpu_sc as plsc

N = 10000
E = 320000
F = 128
NC = 2
NS = 16
NW = NC * NS
EP = E // NW
CHUNK = 80
NCHUNK = EP // CHUNK
GC = 5
GE = GC * CHUNK
NG = NCHUNK // GC
NP = 10240
RPT = NP // NS
ZR = 64
F32 = jnp.float32
_SC_PARAMS = pltpu.CompilerParams(use_tc_tiling_on_sc=False,
                                  needs_layout_passes=False)


def _mesh():
    return plsc.VectorSubcoreMesh(
        core_axis_name="c", subcore_axis_name="s",
        num_cores=NC, num_subcores=NS)


def _zero_f32(ref, nrows, ncols):
    z = jnp.zeros((16,), F32)

    def body(i, _):
        for b in range(ncols // 16):
            ref[i, pl.ds(b * 16, 16)] = z
        return 0

    lax.fori_loop(0, nrows, body, 0)


def _make_agg_kernel(with_deg):
    out_type = [jax.ShapeDtypeStruct((NC, NP, F), F32)]
    scratch = [
        pltpu.VMEM((GE,), jnp.int32),
        pltpu.VMEM((GE,), jnp.int32),
        pltpu.VMEM((GE,), jnp.int32),
        pltpu.VMEM((GE,), jnp.int32),
        pltpu.VMEM((CHUNK, F), F32),
        pltpu.VMEM((CHUNK, F), F32),
        pltpu.VMEM((ZR, F), F32),
        pltpu.VMEM_SHARED((NP, F), F32),
        pltpu.SemaphoreType.DMA,
        pltpu.SemaphoreType.DMA,
        pltpu.SemaphoreType.DMA,
    ]
    if with_deg:
        out_type.append(jax.ShapeDtypeStruct((NC, NP, 16), F32))
        scratch += [
            pltpu.VMEM((CHUNK, 16), F32),
            pltpu.VMEM((ZR, 16), F32),
            pltpu.VMEM_SHARED((NP, 16), F32),
        ]

    def body(table, ei, *refs):
        if with_deg:
            (parts_out, deg_out, gs0, gs1, gd0, gd1, rows0, rows1, zbuf,
             acc_sh, gsem, isem, ssem, ones_v, dbuf, deg_sh) = refs
        else:
            (parts_out, gs0, gs1, gd0, gd1, rows0, rows1, zbuf, acc_sh,
             gsem, isem, ssem) = refs
        rows = (rows0, rows1)
        gsrc = (gs0, gs1)
        gdst = (gd0, gd1)
        c = lax.axis_index("c")
        s = lax.axis_index("s")
        wid = s * NC + c
        ebase = wid * EP

        _zero_f32(zbuf, ZR, F)
        for j in range(RPT // ZR):
            pltpu.sync_copy(zbuf, acc_sh.at[pl.ds(s * RPT + j * ZR, ZR)])
        if with_deg:
            _zero_f32(dbuf, ZR, 16)
            for j in range(RPT // ZR):
                pltpu.sync_copy(dbuf, deg_sh.at[pl.ds(s * RPT + j * ZR, ZR)])
            one = jnp.ones((16,), F32)

            def fill_ones(i, _):
                ones_v[i, :] = one
                return 0

            lax.fori_loop(0, CHUNK, fill_ones, 0)
        plsc.subcore_barrier()

        def fire_idx(g, p):
            pltpu.async_copy(ei.at[0, pl.ds(ebase + g * GE, GE)], gsrc[p],
                             isem)
            pltpu.async_copy(ei.at[1, pl.ds(ebase + g * GE, GE)], gdst[p],
                             isem)

        def drain_idx(p):
            pltpu.make_async_copy(ei.at[0, pl.ds(0, GE)], gsrc[p],
                                  isem).wait()
            pltpu.make_async_copy(ei.at[1, pl.ds(0, GE)], gdst[p],
                                  isem).wait()

        def fire_gather(idx, buf):
            pltpu.async_copy(table.at[idx], buf, gsem)

        def drain_gather(buf):
            pltpu.make_async_copy(table.at[pl.ds(0, CHUNK)], buf, gsem).wait()

        def fire_scatter(idx, buf):
            pltpu.async_copy(buf, acc_sh.at[idx], ssem, add=True)
            if with_deg:
                pltpu.async_copy(ones_v, deg_sh.at[idx], ssem, add=True)

        def drain_scatter():
            pltpu.make_async_copy(
                table.at[pl.ds(0, CHUNK)], rows0, ssem).wait()
            if with_deg:
                pltpu.make_async_copy(
                    deg_out.at[0, pl.ds(0, CHUNK)], ones_v, ssem).wait()

        def emit_group(g, gpar, drain_next, fire_next2, fire_last):
            for i in range(GC):
                kpar = (gpar + i) % 2
                if i == 3 and drain_next:
                    drain_idx(1 - gpar)
                drain_gather(rows[kpar])
                if i == 0:
                    @pl.when(g >= 1)
                    def _():
                        drain_scatter()
                else:
                    drain_scatter()
                if i < GC - 1:
                    fire_gather(gsrc[gpar].at[pl.ds((i + 1) * CHUNK, CHUNK)],
                                rows[1 - kpar])
                elif fire_last:
                    fire_gather(gsrc[1 - gpar].at[pl.ds(0, CHUNK)],
                                rows[1 - kpar])
                fire_scatter(gdst[gpar].at[pl.ds(i * CHUNK, CHUNK)],
                             rows[kpar])
            if fire_next2 == "always":
                fire_idx(g + 2, gpar)
            elif fire_next2 == "guard":
                @pl.when(g + 2 <= NG - 1)
                def _():
                    fire_idx(g + 2, gpar)

        fire_idx(0, 0)
        drain_idx(0)
        fire_idx(1, 1)
        fire_gather(gsrc[0].at[pl.ds(0, CHUNK)], rows0)

        def pair_body(t, _):
            emit_group(2 * t, 0, True, "always", True)
            emit_group(2 * t + 1, 1, True, "guard", True)
            return 0

        lax.fori_loop(0, (NG - 1) // 2, pair_body, 0)
        emit_group(NG - 1, 0, False, None, False)
        drain_scatter()
        plsc.subcore_barrier()

        pltpu.sync_copy(acc_sh.at[pl.ds(s * RPT, RPT)],
                        parts_out.at[c, pl.ds(s * RPT, RPT)])
        if with_deg:
            pltpu.sync_copy(deg_sh.at[pl.ds(s * RPT, RPT)],
                            deg_out.at[c, pl.ds(s * RPT, RPT)])

    return pl.kernel(body, tuple(out_type), mesh=_mesh(),
                     scratch_types=scratch, compiler_params=_SC_PARAMS)


@functools.lru_cache(maxsize=None)
def _get_agg_kernel(with_deg):
    return _make_agg_kernel(with_deg)


def _predictor_sc(A, B, pei, wp2):
    def body(a_hbm, b_hbm, pei, w_hbm, out,
             u_v, v_v, ra0, ra1, rb0, rb1, w_v, s0, s1, gsem, ssem):
        ras = (ra0, ra1)
        rbs = (rb0, rb1)
        s16 = (s0, s1)
        c = lax.axis_index("c")
        s = lax.axis_index("s")
        wid = s * NC + c
        ebase = wid * EP
        pltpu.sync_copy(pei.at[0, pl.ds(ebase, EP)], u_v)
        pltpu.sync_copy(pei.at[1, pl.ds(ebase, EP)], v_v)
        pltpu.sync_copy(w_hbm, w_v)
        wbs = [w_v[pl.ds(b * 16, 16)] for b in range(F // 16)]

        def fire_gathers(k, ra, rb):
            pltpu.async_copy(
                a_hbm.at[u_v.at[pl.ds(k * CHUNK, CHUNK)]], ra, gsem)
            pltpu.async_copy(
                b_hbm.at[v_v.at[pl.ds(k * CHUNK, CHUNK)]], rb, gsem)

        def drain_gathers(ra, rb):
            pltpu.make_async_copy(a_hbm.at[pl.ds(0, CHUNK)], ra, gsem).wait()
            pltpu.make_async_copy(b_hbm.at[pl.ds(0, CHUNK)], rb, gsem).wait()

        def compute(ra, rb, sbuf):
            @plsc.parallel_loop(0, CHUNK, step=1, unroll=4)
            def edge_body(e):
                acc = jnp.zeros((16,), F32)
                for b in range(F // 16):
                    av = ra[e, pl.ds(b * 16, 16)]
                    bv = rb[e, pl.ds(b * 16, 16)]
                    acc = acc + jnp.maximum(av + bv, 0.0) * wbs[b]
                sbuf[pl.ds(e * 16, 16)] = acc

        def drain_store(sbuf):
            pltpu.make_async_copy(
                out.at[pl.ds(0, CHUNK * 16)], sbuf, ssem).wait()

        fire_gathers(0, ra0, rb0)

        def chunk_body(j, _):
            for h in range(2):
                k = 2 * j + h
                drain_gathers(ras[h], rbs[h])
                fire_gathers(k + 1, ras[1 - h], rbs[1 - h])

                @pl.when(k >= 2)
                def _():
                    drain_store(s16[h])

                compute(ras[h], rbs[h], s16[h])
                pltpu.async_copy(
                    s16[h],
                    out.at[pl.ds((ebase + k * CHUNK) * 16, CHUNK * 16)],
                    ssem)
            return 0

        lax.fori_loop(0, (NCHUNK - 1) // 2, chunk_body, 0)
        k = NCHUNK - 1
        drain_gathers(ras[0], rbs[0])
        drain_store(s16[0])
        compute(ras[0], rbs[0], s16[0])
        drain_store(s16[1])
        pltpu.sync_copy(s16[0],
                        out.at[pl.ds((ebase + k * CHUNK) * 16, CHUNK * 16)])

    return pl.kernel(
        body,
        jax.ShapeDtypeStruct((E * 16,), F32),
        mesh=_mesh(),
        scratch_types=[
            pltpu.VMEM((EP,), jnp.int32),
            pltpu.VMEM((EP,), jnp.int32),
            pltpu.VMEM((CHUNK, F), F32),
            pltpu.VMEM((CHUNK, F), F32),
            pltpu.VMEM((CHUNK, F), F32),
            pltpu.VMEM((CHUNK, F), F32),
            pltpu.VMEM((F,), F32),
            pltpu.VMEM((CHUNK * 16,), F32),
            pltpu.VMEM((CHUNK * 16,), F32),
            pltpu.SemaphoreType.DMA,
            pltpu.SemaphoreType.DMA,
        ],
        compiler_params=_SC_PARAMS,
    )(A, B, pei, wp2)


_RB = 1000
_RBP = 1024


def _mm_body(x_ref, w_ref, o_ref):
    o_ref[...] = jnp.dot(x_ref[...], w_ref[...], preferred_element_type=F32)


def _tc_matmul(x, w):
    return pl.pallas_call(
        _mm_body,
        grid=(N // _RB,),
        in_specs=[
            pl.BlockSpec((_RB, F), lambda i: (i, 0)),
            pl.BlockSpec((F, F), lambda i: (0, 0)),
        ],
        out_specs=pl.BlockSpec((_RB, F), lambda i: (i, 0)),
        out_shape=jax.ShapeDtypeStruct((N, F), F32),
    )(x, w)


def _deg_from16(d_ref):
    deg = jnp.sum(d_ref[...], axis=(0, 2)) * (1.0 / 16.0)
    return jnp.maximum(deg, 1.0)


def _lay1_body(p_ref, d_ref, b1_ref, w2_ref, o_ref):
    acc = p_ref[0] + p_ref[1]
    deg = _deg_from16(d_ref)
    h = jnp.maximum(acc / deg[:, None] + b1_ref[...], 0.0)
    o_ref[...] = jnp.dot(h, w2_ref[...], preferred_element_type=F32)


def _tc_layer1(parts, deg16, b1, w2):
    return pl.pallas_call(
        _lay1_body,
        grid=(NP // _RBP,),
        in_specs=[
            pl.BlockSpec((NC, _RBP, F), lambda i: (0, i, 0)),
            pl.BlockSpec((NC, _RBP, 16), lambda i: (0, i, 0)),
            pl.BlockSpec((1, F), lambda i: (0, 0)),
            pl.BlockSpec((F, F), lambda i: (0, 0)),
        ],
        out_specs=pl.BlockSpec((_RBP, F), lambda i: (i, 0)),
        out_shape=jax.ShapeDtypeStruct((NP, F), F32),
    )(parts, deg16, b1, w2)


def _lay2_body(p_ref, d_ref, b2_ref, wp1_ref, bp1_ref, a_ref, b_ref):
    acc = p_ref[0] + p_ref[1]
    deg = _deg_from16(d_ref)
    h2 = acc / deg[:, None] + b2_ref[...]
    a_ref[...] = (jnp.dot(h2, wp1_ref[0:F, :], preferred_element_type=F32)
                  + bp1_ref[...])
    b_ref[...] = jnp.dot(h2, wp1_ref[F:2 * F, :], preferred_element_type=F32)


def _tc_layer2(parts, deg16, b2, wp1, bp1):
    return pl.pallas_call(
        _lay2_body,
        grid=(NP // _RBP,),
        in_specs=[
            pl.BlockSpec((NC, _RBP, F), lambda i: (0, i, 0)),
            pl.BlockSpec((NC, _RBP, 16), lambda i: (0, i, 0)),
            pl.BlockSpec((1, F), lambda i: (0, 0)),
            pl.BlockSpec((2 * F, F), lambda i: (0, 0)),
            pl.BlockSpec((1, F), lambda i: (0, 0)),
        ],
        out_specs=[
            pl.BlockSpec((_RBP, F), lambda i: (i, 0)),
            pl.BlockSpec((_RBP, F), lambda i: (i, 0)),
        ],
        out_shape=[
            jax.ShapeDtypeStruct((NP, F), F32),
            jax.ShapeDtypeStruct((NP, F), F32),
        ],
    )(parts, deg16, b2, wp1, bp1)


_ER = E * 16 // F
_EB = 4000


def _red_body(s_ref, bp2_ref, o_ref):
    m = (jnp.arange(F)[:, None] // 16 == jnp.arange(8)[None, :]).astype(F32)
    o_ref[...] = (jnp.dot(s_ref[...], m, preferred_element_type=F32)
                  + bp2_ref[0, 0])


def _tc_reduce(s16, bp2):
    return pl.pallas_call(
        _red_body,
        grid=(_ER // _EB,),
        in_specs=[
            pl.BlockSpec((_EB, F), lambda i: (i, 0)),
            pl.BlockSpec((1, 1), lambda i: (0, 0)),
        ],
        out_specs=pl.BlockSpec((_EB, 8), lambda i: (i, 0)),
        out_shape=jax.ShapeDtypeStruct((_ER, 8), F32),
    )(s16, bp2)


def kernel(x, edge_index, pred_edge_index, W1, b1, W2, b2, Wp1, bp1, Wp2, bp2):
    ei = edge_index.astype(jnp.int32)
    pei = pred_edge_index.astype(jnp.int32)

    xw = _tc_matmul(x, W1)
    parts1, deg16 = _get_agg_kernel(True)(xw, ei)
    hw = _tc_layer1(parts1, deg16, b1.reshape(1, F), W2)
    (parts2,) = _get_agg_kernel(False)(hw, ei)
    A, B = _tc_layer2(parts2, deg16, b2.reshape(1, F), Wp1, bp1.reshape(1, F))
    s16 = _predictor_sc(A, B, pei, Wp2.reshape(F)).reshape(_ER, F)
    return _tc_reduce(s16, bp2.reshape(1, 1)).reshape(E, 1)
